# baseline (speedup 1.0000x reference)
import functools

import jax
import jax.numpy as jnp
from jax import lax
from jax.experimental import pallas as pl
from jax.experimental.pallas import tpu as pltpu
from jax.experimental.pallas import tpu_sc as plsc

N_NODES = 10000
N_EDGES = 320000
D = 128

NC = 2
NS = 16
NW = NC * NS

CHUNK = 64
N_CHUNKS = 156
E_PER_TILE = N_CHUNKS * CHUNK
EXTRA_BASE = NW * E_PER_TILE
NB = 4
NP = 10240
ROWS_PER_TILE = NP // NS


def _sc_segment_sum(src, dst, feat):
    mesh = plsc.VectorSubcoreMesh(core_axis_name="c", subcore_axis_name="s")

    @functools.partial(
        pl.kernel,
        out_type=[
            jax.ShapeDtypeStruct((NC * NP, D), jnp.float32),
            jax.ShapeDtypeStruct((NC, NP // D, D), jnp.float32),
        ],
        mesh=mesh,
        compiler_params=pltpu.CompilerParams(needs_layout_passes=False),
        scratch_types=(
            [pltpu.VMEM((CHUNK,), jnp.int32)] * NB
            + [pltpu.VMEM((CHUNK,), jnp.int32)] * NB
            + [pltpu.VMEM((CHUNK, D), jnp.float32)] * NB
            + [
                pltpu.VMEM((NP // D, D), jnp.float32),
                pltpu.VMEM_SHARED((NP, D), jnp.float32),
                pltpu.VMEM_SHARED((NP // D, D), jnp.float32),
                pltpu.VMEM((NP // D,), jnp.int32),
            ]
            + [pltpu.SemaphoreType.DMA] * (2 * NB + 1)
        ),
    )
    def seg(src_hbm, dst_hbm, feat_hbm, acc_out, deg_out, *scr):
        sidxs = scr[0:NB]
        didxs = scr[NB:2 * NB]
        rowbufs = scr[2 * NB:3 * NB]
        deg_local = scr[3 * NB]
        acc_sh = scr[3 * NB + 1]
        deg_sh = scr[3 * NB + 2]
        rowiota = scr[3 * NB + 3]
        gsems = scr[3 * NB + 4:4 * NB + 4]
        isems = scr[4 * NB + 4:5 * NB + 4]
        sem = scr[5 * NB + 4]
        sidx, didx, rows = sidxs[0], didxs[0], rowbufs[0]
        c = lax.axis_index("c")
        s = lax.axis_index("s")
        wid = s * NC + c

        def fill_iota(buf, start):
            for k in range(CHUNK // 16):
                buf[pl.ds(k * 16, 16)] = start + k * 16 + lax.iota(jnp.int32, 16)

        def fill_rows(i, _):
            for j in range(D // 16):
                rows[i, pl.ds(j * 16, 16)] = jnp.zeros((16,), jnp.float32)
            return 0

        lax.fori_loop(0, CHUNK, fill_rows, 0)

        def fill_deg(i, _):
            for k in range(8):
                deg_local[i, pl.ds(k * 16, 16)] = jnp.zeros(
                    (16,), jnp.float32)
            return 0

        lax.fori_loop(0, NP // D, fill_deg, 0)
        for k in range(NP // D // 16):
            rowiota[pl.ds(k * 16, 16)] = k * 16 + lax.iota(jnp.int32, 16)

        @pl.when(s == 0)
        def _():
            pltpu.sync_copy(deg_local, deg_sh)

        base = s * ROWS_PER_TILE
        nz = ROWS_PER_TILE // CHUNK
        for j in range(nz):
            b = j % NB
            if j >= NB:
                pltpu.make_async_copy(rows, acc_sh.at[sidxs[b]],
                                      gsems[b]).wait()
            fill_iota(sidxs[b], base + j * CHUNK)
            pltpu.async_copy(rows, acc_sh.at[sidxs[b]], gsems[b])
        for j in range(nz - NB, nz):
            b = j % NB
            pltpu.make_async_copy(rows, acc_sh.at[sidxs[b]], gsems[b]).wait()
        plsc.subcore_barrier()

        e0 = wid * E_PER_TILE
        ones16 = jnp.ones((16,), jnp.float32)

        def load_idx(b, i):
            eb = e0 + i * CHUNK
            pltpu.async_copy(src_hbm.at[pl.ds(eb, CHUNK)], sidxs[b], isems[b])
            pltpu.async_copy(dst_hbm.at[pl.ds(eb, CHUNK)], didxs[b], isems[b])

        def wait_idx(b, i):
            eb = e0 + i * CHUNK
            pltpu.make_async_copy(src_hbm.at[pl.ds(eb, CHUNK)], sidxs[b],
                                  isems[b]).wait()
            pltpu.make_async_copy(dst_hbm.at[pl.ds(eb, CHUNK)], didxs[b],
                                  isems[b]).wait()

        def issue_gather(b):
            pltpu.async_copy(feat_hbm.at[sidxs[b]], rowbufs[b], gsems[b])

        def consume_core(b, issue_next):
            pltpu.make_async_copy(feat_hbm.at[sidxs[b]], rowbufs[b],
                                  gsems[b]).wait()
            d = pltpu.async_copy(rowbufs[b], acc_sh.at[didxs[b]], sem,
                                 add=True)
            issue_next()
            for k in range(CHUNK // 16):
                dv = didxs[b][pl.ds(k * 16, 16)]
                plsc.addupdate_scatter(deg_local,
                                       [dv >> 7, dv & 127], ones16)
            d.wait()

        GD = NB - 1
        for b in range(NB):
            load_idx(b, b)
        for b in range(GD):
            wait_idx(b, b)
            issue_gather(b)

        def pipe_body(k, _):
            for b in range(NB):
                i = k * NB + b
                i2 = i + GD
                b2 = (b + GD) % NB

                def issue_next():
                    @pl.when(i2 < N_CHUNKS)
                    def _():
                        wait_idx(b2, i2)
                        issue_gather(b2)

                consume_core(b, issue_next)
                i3 = i + NB

                @pl.when(i3 < N_CHUNKS)
                def _():
                    load_idx(b, i3)
            return 0

        lax.fori_loop(0, N_CHUNKS // NB, pipe_body, 0)
        for i in range((N_CHUNKS // NB) * NB, N_CHUNKS):
            consume_core(i % NB, lambda: None)

        @pl.when(wid < (N_EDGES - EXTRA_BASE) // CHUNK)
        def _():
            eb = EXTRA_BASE + wid * CHUNK
            pltpu.sync_copy(src_hbm.at[pl.ds(eb, CHUNK)], sidxs[0])
            pltpu.sync_copy(dst_hbm.at[pl.ds(eb, CHUNK)], didxs[0])
            pltpu.async_copy(feat_hbm.at[sidxs[0]], rowbufs[0],
                             gsems[0]).wait()
            d = pltpu.async_copy(rowbufs[0], acc_sh.at[didxs[0]], sem,
                                 add=True)
            for k in range(CHUNK // 16):
                dv = didxs[0][pl.ds(k * 16, 16)]
                plsc.addupdate_scatter(deg_local,
                                       [dv >> 7, dv & 127], ones16)
            d.wait()

        pltpu.sync_copy(deg_local, deg_sh.at[rowiota], add=True)
        plsc.subcore_barrier()

        @pl.when(s == 0)
        def _():
            pltpu.sync_copy(acc_sh, acc_out.at[pl.ds(c * NP, NP)])
            pltpu.sync_copy(deg_sh, deg_out.at[c])

    return seg(src, dst, feat)


BLK = 1000


def _tc_body(feat_ref, acc_ref, deg_ref, wnT_ref, wsT_ref, b_ref, out_ref):
    deg = deg_ref[:, 0:1] + deg_ref[:, 1:2]
    scale = jnp.where(deg > 0, 1.0 / jnp.maximum(deg, 1.0), 0.0)
    neigh = (acc_ref[0] + acc_ref[1]) * scale
    out_ref[...] = (
        jnp.dot(feat_ref[...], wsT_ref[...],
                preferred_element_type=jnp.float32,
                precision=lax.Precision.HIGHEST)
        + b_ref[...]
        + jnp.dot(neigh, wnT_ref[...],
                  preferred_element_type=jnp.float32,
                  precision=lax.Precision.HIGHEST)
    )


def _tc_combine(feat, acc, deg, wnT, wsT, b):
    return pl.pallas_call(
        _tc_body,
        grid=(N_NODES // BLK,),
        in_specs=[
            pl.BlockSpec((BLK, D), lambda i: (i, 0)),
            pl.BlockSpec((NC, BLK, D), lambda i: (0, i, 0)),
            pl.BlockSpec((BLK, NC), lambda i: (i, 0)),
            pl.BlockSpec((D, D), lambda i: (0, 0)),
            pl.BlockSpec((D, D), lambda i: (0, 0)),
            pl.BlockSpec((1, D), lambda i: (0, 0)),
        ],
        out_specs=pl.BlockSpec((BLK, D), lambda i: (i, 0)),
        out_shape=jax.ShapeDtypeStruct((N_NODES, D), jnp.float32),
    )(feat, acc, deg, wnT, wsT, b)


def kernel(feat, edge_index, W_neigh, W_self, b_self):
    src = edge_index[0].astype(jnp.int32)
    dst = edge_index[1].astype(jnp.int32)
    acc_flat, deg = _sc_segment_sum(src, dst, feat)
    acc = acc_flat.reshape(NC, NP, D)
    deg2 = deg.reshape(NC, NP)
    return _tc_combine(feat, acc, deg2.T, W_neigh.T, W_self.T,
                       b_self.reshape(1, D))


# speedup vs baseline: 1.0049x; 1.0049x over previous
import functools

import jax
import jax.numpy as jnp
from jax import lax
from jax.experimental import pallas as pl
from jax.experimental.pallas import tpu as pltpu
from jax.experimental.pallas import tpu_sc as plsc

N_NODES = 10000
N_EDGES = 320000
D = 128

NC = 2
NS = 16
NW = NC * NS

CHUNK = 64
N_CHUNKS = 156
E_PER_TILE = N_CHUNKS * CHUNK
EXTRA_BASE = NW * E_PER_TILE
NB = 4
NP = 10240
ROWS_PER_TILE = NP // NS


def _sc_segment_sum(src, dst, feat):
    mesh = plsc.VectorSubcoreMesh(core_axis_name="c", subcore_axis_name="s")

    @functools.partial(
        pl.kernel,
        out_type=[
            jax.ShapeDtypeStruct((NC * NP, D), jnp.float32),
            jax.ShapeDtypeStruct((NW, NP), jnp.float32),
        ],
        mesh=mesh,
        compiler_params=pltpu.CompilerParams(needs_layout_passes=False),
        scratch_types=(
            [pltpu.VMEM((CHUNK,), jnp.int32)] * NB
            + [pltpu.VMEM((CHUNK,), jnp.int32)] * NB
            + [pltpu.VMEM((CHUNK, D), jnp.float32)] * NB
            + [
                pltpu.VMEM((NP,), jnp.float32),
                pltpu.VMEM_SHARED((NP, D), jnp.float32),
            ]
            + [pltpu.SemaphoreType.DMA] * (2 * NB + 1)
        ),
    )
    def seg(src_hbm, dst_hbm, feat_hbm, acc_out, deg_out, *scr):
        sidxs = scr[0:NB]
        didxs = scr[NB:2 * NB]
        rowbufs = scr[2 * NB:3 * NB]
        deg_local = scr[3 * NB]
        acc_sh = scr[3 * NB + 1]
        gsems = scr[3 * NB + 2:4 * NB + 2]
        isems = scr[4 * NB + 2:5 * NB + 2]
        sem = scr[5 * NB + 2]
        sidx, didx, rows = sidxs[0], didxs[0], rowbufs[0]
        c = lax.axis_index("c")
        s = lax.axis_index("s")
        wid = s * NC + c

        def fill_iota(buf, start):
            for k in range(CHUNK // 16):
                buf[pl.ds(k * 16, 16)] = start + k * 16 + lax.iota(jnp.int32, 16)

        def fill_rows(i, _):
            for j in range(D // 16):
                rows[i, pl.ds(j * 16, 16)] = jnp.zeros((16,), jnp.float32)
            return 0

        lax.fori_loop(0, CHUNK, fill_rows, 0)

        def fill_deg(i, _):
            for k in range(8):
                deg_local[pl.ds(i * 128 + k * 16, 16)] = jnp.zeros(
                    (16,), jnp.float32)
            return 0

        lax.fori_loop(0, NP // 128, fill_deg, 0)

        base = s * ROWS_PER_TILE
        nz = ROWS_PER_TILE // CHUNK
        for j in range(nz):
            b = j % NB
            if j >= NB:
                pltpu.make_async_copy(rows, acc_sh.at[sidxs[b]],
                                      gsems[b]).wait()
            fill_iota(sidxs[b], base + j * CHUNK)
            pltpu.async_copy(rows, acc_sh.at[sidxs[b]], gsems[b])
        for j in range(nz - NB, nz):
            b = j % NB
            pltpu.make_async_copy(rows, acc_sh.at[sidxs[b]], gsems[b]).wait()
        plsc.subcore_barrier()

        e0 = wid * E_PER_TILE
        ones16 = jnp.ones((16,), jnp.float32)

        def load_idx(b, i):
            eb = e0 + i * CHUNK
            pltpu.async_copy(src_hbm.at[pl.ds(eb, CHUNK)], sidxs[b], isems[b])
            pltpu.async_copy(dst_hbm.at[pl.ds(eb, CHUNK)], didxs[b], isems[b])

        def wait_idx(b, i):
            eb = e0 + i * CHUNK
            pltpu.make_async_copy(src_hbm.at[pl.ds(eb, CHUNK)], sidxs[b],
                                  isems[b]).wait()
            pltpu.make_async_copy(dst_hbm.at[pl.ds(eb, CHUNK)], didxs[b],
                                  isems[b]).wait()

        def issue_gather(b):
            pltpu.async_copy(feat_hbm.at[sidxs[b]], rowbufs[b], gsems[b])

        def consume_core(b, issue_next):
            pltpu.make_async_copy(feat_hbm.at[sidxs[b]], rowbufs[b],
                                  gsems[b]).wait()
            d = pltpu.async_copy(rowbufs[b], acc_sh.at[didxs[b]], sem,
                                 add=True)
            issue_next()
            for k in range(CHUNK // 16):
                dv = didxs[b][pl.ds(k * 16, 16)]
                plsc.addupdate_scatter(deg_local, [dv], ones16)
            d.wait()

        GD = NB - 1
        for b in range(NB):
            load_idx(b, b)
        for b in range(GD):
            wait_idx(b, b)
            issue_gather(b)

        def pipe_body(k, _):
            for b in range(NB):
                i = k * NB + b
                i2 = i + GD
                b2 = (b + GD) % NB

                def issue_next():
                    @pl.when(i2 < N_CHUNKS)
                    def _():
                        wait_idx(b2, i2)
                        issue_gather(b2)

                consume_core(b, issue_next)
                i3 = i + NB

                @pl.when(i3 < N_CHUNKS)
                def _():
                    load_idx(b, i3)
            return 0

        lax.fori_loop(0, N_CHUNKS // NB, pipe_body, 0)
        for i in range((N_CHUNKS // NB) * NB, N_CHUNKS):
            consume_core(i % NB, lambda: None)

        @pl.when(wid < (N_EDGES - EXTRA_BASE) // CHUNK)
        def _():
            eb = EXTRA_BASE + wid * CHUNK
            pltpu.sync_copy(src_hbm.at[pl.ds(eb, CHUNK)], sidxs[0])
            pltpu.sync_copy(dst_hbm.at[pl.ds(eb, CHUNK)], didxs[0])
            pltpu.async_copy(feat_hbm.at[sidxs[0]], rowbufs[0],
                             gsems[0]).wait()
            d = pltpu.async_copy(rowbufs[0], acc_sh.at[didxs[0]], sem,
                                 add=True)
            for k in range(CHUNK // 16):
                dv = didxs[0][pl.ds(k * 16, 16)]
                plsc.addupdate_scatter(deg_local, [dv], ones16)
            d.wait()

        plsc.subcore_barrier()

        pltpu.sync_copy(deg_local, deg_out.at[wid])

        @pl.when(s == 0)
        def _():
            pltpu.sync_copy(acc_sh, acc_out.at[pl.ds(c * NP, NP)])

    return seg(src, dst, feat)


BLK = 1000


def _tc_body(feat_ref, acc_ref, deg_ref, wnT_ref, wsT_ref, b_ref, out_ref):
    deg = jnp.sum(deg_ref[...], axis=1)[:, None]
    scale = jnp.where(deg > 0, 1.0 / jnp.maximum(deg, 1.0), 0.0)
    neigh = (acc_ref[0] + acc_ref[1]) * scale
    out_ref[...] = (
        jnp.dot(feat_ref[...], wsT_ref[...],
                preferred_element_type=jnp.float32,
                precision=lax.Precision.HIGHEST)
        + b_ref[...]
        + jnp.dot(neigh, wnT_ref[...],
                  preferred_element_type=jnp.float32,
                  precision=lax.Precision.HIGHEST)
    )


def _tc_combine(feat, acc, deg, wnT, wsT, b):
    return pl.pallas_call(
        _tc_body,
        grid=(N_NODES // BLK,),
        in_specs=[
            pl.BlockSpec((BLK, D), lambda i: (i, 0)),
            pl.BlockSpec((NC, BLK, D), lambda i: (0, i, 0)),
            pl.BlockSpec((BLK, NW), lambda i: (i, 0)),
            pl.BlockSpec((D, D), lambda i: (0, 0)),
            pl.BlockSpec((D, D), lambda i: (0, 0)),
            pl.BlockSpec((1, D), lambda i: (0, 0)),
        ],
        out_specs=pl.BlockSpec((BLK, D), lambda i: (i, 0)),
        out_shape=jax.ShapeDtypeStruct((N_NODES, D), jnp.float32),
    )(feat, acc, deg, wnT, wsT, b)


def kernel(feat, edge_index, W_neigh, W_self, b_self):
    src = edge_index[0].astype(jnp.int32)
    dst = edge_index[1].astype(jnp.int32)
    acc_flat, deg = _sc_segment_sum(src, dst, feat)
    acc = acc_flat.reshape(NC, NP, D)
    return _tc_combine(feat, acc, deg.T, W_neigh.T, W_self.T,
                       b_self.reshape(1, D))


# speedup vs baseline: 1.0061x; 1.0012x over previous
import functools

import jax
import jax.numpy as jnp
from jax import lax
from jax.experimental import pallas as pl
from jax.experimental.pallas import tpu as pltpu
from jax.experimental.pallas import tpu_sc as plsc

N_NODES = 10000
N_EDGES = 320000
D = 128

NC = 2
NS = 16
NW = NC * NS

CHUNK = 64
N_CHUNKS = 156
E_PER_TILE = N_CHUNKS * CHUNK
EXTRA_BASE = NW * E_PER_TILE
NB = 4
NP = 10240
ROWS_PER_TILE = NP // NS


def _sc_segment_sum(src, dst, feat):
    mesh = plsc.VectorSubcoreMesh(core_axis_name="c", subcore_axis_name="s")

    @functools.partial(
        pl.kernel,
        out_type=[
            jax.ShapeDtypeStruct((NC * NP, D), jnp.float32),
            jax.ShapeDtypeStruct((NW, NP), jnp.float32),
        ],
        mesh=mesh,
        compiler_params=pltpu.CompilerParams(needs_layout_passes=False),
        scratch_types=(
            [pltpu.VMEM((CHUNK,), jnp.int32)] * NB
            + [pltpu.VMEM((CHUNK,), jnp.int32)] * NB
            + [pltpu.VMEM((CHUNK, D), jnp.float32)] * NB
            + [
                pltpu.VMEM((NP,), jnp.float32),
                pltpu.VMEM_SHARED((NP, D), jnp.float32),
            ]
            + [pltpu.SemaphoreType.DMA] * (2 * NB + 1)
        ),
    )
    def seg(src_hbm, dst_hbm, feat_hbm, acc_out, deg_out, *scr):
        sidxs = scr[0:NB]
        didxs = scr[NB:2 * NB]
        rowbufs = scr[2 * NB:3 * NB]
        deg_local = scr[3 * NB]
        acc_sh = scr[3 * NB + 1]
        gsems = scr[3 * NB + 2:4 * NB + 2]
        isems = scr[4 * NB + 2:5 * NB + 2]
        sem = scr[5 * NB + 2]
        rows = rowbufs[0]
        c = lax.axis_index("c")
        s = lax.axis_index("s")
        wid = s * NC + c

        def fill_iota(buf, start):
            for k in range(CHUNK // 16):
                buf[pl.ds(k * 16, 16)] = start + k * 16 + lax.iota(jnp.int32, 16)

        def fill_rows(i, _):
            for j in range(D // 16):
                rows[i, pl.ds(j * 16, 16)] = jnp.zeros((16,), jnp.float32)
            return 0

        lax.fori_loop(0, CHUNK, fill_rows, 0)

        def fill_deg(i, _):
            for k in range(8):
                deg_local[pl.ds(i * 128 + k * 16, 16)] = jnp.zeros(
                    (16,), jnp.float32)
            return 0

        lax.fori_loop(0, NP // 128, fill_deg, 0)

        base = s * ROWS_PER_TILE
        nz = ROWS_PER_TILE // CHUNK
        for j in range(nz):
            b = j % NB
            if j >= NB:
                pltpu.make_async_copy(rows, acc_sh.at[sidxs[b]],
                                      gsems[b]).wait()
            fill_iota(sidxs[b], base + j * CHUNK)
            pltpu.async_copy(rows, acc_sh.at[sidxs[b]], gsems[b])
        for j in range(nz - NB, nz):
            b = j % NB
            pltpu.make_async_copy(rows, acc_sh.at[sidxs[b]], gsems[b]).wait()
        plsc.subcore_barrier()

        e0 = wid * E_PER_TILE
        ones16 = jnp.ones((16,), jnp.float32)

        def load_idx(b, i):
            eb = e0 + i * CHUNK
            pltpu.async_copy(src_hbm.at[pl.ds(eb, CHUNK)], sidxs[b], isems[b])
            pltpu.async_copy(dst_hbm.at[pl.ds(eb, CHUNK)], didxs[b], isems[b])

        def wait_idx(b, i):
            eb = e0 + i * CHUNK
            pltpu.make_async_copy(src_hbm.at[pl.ds(eb, CHUNK)], sidxs[b],
                                  isems[b]).wait()
            pltpu.make_async_copy(dst_hbm.at[pl.ds(eb, CHUNK)], didxs[b],
                                  isems[b]).wait()

        def issue_gather(b):
            pltpu.async_copy(feat_hbm.at[sidxs[b]], rowbufs[b], gsems[b])

        def consume_core(b, issue_next):
            pltpu.make_async_copy(feat_hbm.at[sidxs[b]], rowbufs[b],
                                  gsems[b]).wait()
            d = pltpu.async_copy(rowbufs[b], acc_sh.at[didxs[b]], sem,
                                 add=True)
            issue_next()
            for k in range(CHUNK // 16):
                dv = didxs[b][pl.ds(k * 16, 16)]
                plsc.addupdate_scatter(deg_local, [dv], ones16)
            d.wait()

        GD = NB - 1
        for b in range(NB):
            load_idx(b, b)
        for b in range(GD):
            wait_idx(b, b)
            issue_gather(b)

        def pipe_body(k, _):
            for b in range(NB):
                i = k * NB + b
                i2 = i + GD
                b2 = (b + GD) % NB

                def issue_next():
                    @pl.when(i2 < N_CHUNKS)
                    def _():
                        wait_idx(b2, i2)
                        issue_gather(b2)

                consume_core(b, issue_next)
                i3 = i + NB

                @pl.when(i3 < N_CHUNKS)
                def _():
                    load_idx(b, i3)
            return 0

        lax.fori_loop(0, N_CHUNKS // NB, pipe_body, 0)
        for i in range((N_CHUNKS // NB) * NB, N_CHUNKS):
            consume_core(i % NB, lambda: None)

        @pl.when(wid < (N_EDGES - EXTRA_BASE) // CHUNK)
        def _():
            eb = EXTRA_BASE + wid * CHUNK
            pltpu.sync_copy(src_hbm.at[pl.ds(eb, CHUNK)], sidxs[0])
            pltpu.sync_copy(dst_hbm.at[pl.ds(eb, CHUNK)], didxs[0])
            pltpu.async_copy(feat_hbm.at[sidxs[0]], rowbufs[0],
                             gsems[0]).wait()
            d = pltpu.async_copy(rowbufs[0], acc_sh.at[didxs[0]], sem,
                                 add=True)
            for k in range(CHUNK // 16):
                dv = didxs[0][pl.ds(k * 16, 16)]
                plsc.addupdate_scatter(deg_local, [dv], ones16)
            d.wait()

        plsc.subcore_barrier()

        pltpu.sync_copy(deg_local, deg_out.at[wid])

        @pl.when(s == 0)
        def _():
            pltpu.sync_copy(acc_sh, acc_out.at[pl.ds(c * NP, NP)])

    return seg(src, dst, feat)


BLK = 1000


def _tc_body(feat_ref, acc_ref, deg_ref, wnT_ref, wsT_ref, b_ref, out_ref):
    deg = jnp.sum(deg_ref[...], axis=1)[:, None]
    scale = jnp.where(deg > 0, 1.0 / jnp.maximum(deg, 1.0), 0.0)
    neigh = (acc_ref[0] + acc_ref[1]) * scale
    out_ref[...] = (
        jnp.dot(feat_ref[...], wsT_ref[...],
                preferred_element_type=jnp.float32,
                precision=lax.Precision.HIGHEST)
        + b_ref[...]
        + jnp.dot(neigh, wnT_ref[...],
                  preferred_element_type=jnp.float32,
                  precision=lax.Precision.HIGHEST)
    )


def _tc_combine(feat, acc, deg, wnT, wsT, b):
    return pl.pallas_call(
        _tc_body,
        grid=(N_NODES // BLK,),
        in_specs=[
            pl.BlockSpec((BLK, D), lambda i: (i, 0)),
            pl.BlockSpec((NC, BLK, D), lambda i: (0, i, 0)),
            pl.BlockSpec((BLK, NW), lambda i: (i, 0)),
            pl.BlockSpec((D, D), lambda i: (0, 0)),
            pl.BlockSpec((D, D), lambda i: (0, 0)),
            pl.BlockSpec((1, D), lambda i: (0, 0)),
        ],
        out_specs=pl.BlockSpec((BLK, D), lambda i: (i, 0)),
        out_shape=jax.ShapeDtypeStruct((N_NODES, D), jnp.float32),
    )(feat, acc, deg, wnT, wsT, b)


def kernel(feat, edge_index, W_neigh, W_self, b_self):
    src = edge_index[0].astype(jnp.int32)
    dst = edge_index[1].astype(jnp.int32)
    acc_flat, deg = _sc_segment_sum(src, dst, feat)
    acc = acc_flat.reshape(NC, NP, D)
    return _tc_combine(feat, acc, deg.T, W_neigh.T, W_self.T,
                       b_self.reshape(1, D))
